# baseline (speedup 1.0000x reference)
import functools

import jax
import jax.numpy as jnp
import numpy as np
from jax import lax
from jax.experimental import pallas as pl
from jax.experimental.pallas import tpu as pltpu
from jax.experimental.pallas import tpu_sc as plsc

_SCALE = 10.0
_B = 64
_C = 100000
_NEG = int(0.01 * (_C - 1))
_RANK = _NEG + 1
_NC = 2
_NS = 16
_L = 16
_NW = _NC * _NS
_RPW = _B // _NW
_NV = _C // _L
_HI = np.uint32(0x80000000)


def _bc(x, dt):
    return lax.bitcast_convert_type(x, dt)


def _vfull(val, dt):
    return jnp.full((16,), val, dt)


def _sortable(v):
    bu = _bc(v, jnp.uint32)
    asr = lax.shift_right_arithmetic(_bc(v, jnp.int32), _vfull(31, jnp.int32))
    flip = _bc(asr, jnp.uint32) | _vfull(_HI, jnp.uint32)
    return bu ^ flip


def _unsortable(k):
    b = jnp.where(k >= _vfull(_HI, jnp.uint32), k ^ _vfull(_HI, jnp.uint32), ~k)
    return _bc(b, jnp.float32)


def _sc_body(x_hbm, t_hbm, out_hbm, row, hist, tvm, outb):
    cid = lax.axis_index("c")
    sid = lax.axis_index("s")
    wid = sid * _NC + cid
    lane = lax.iota(jnp.int32, 16)
    fones = jnp.ones((16,), jnp.float32)
    fzeros = jnp.zeros((16,), jnp.float32)

    pltpu.sync_copy(t_hbm, tvm)

    def zb(i, _):
        hist[pl.ds(i * 16, 16)] = fzeros
        return jnp.int32(0)

    outv = jnp.zeros((16,), jnp.float32)

    for j in range(_RPW):
        r = wid * _RPW + j
        pltpu.sync_copy(x_hbm.at[r], row)

        lax.fori_loop(0, 256, zb, jnp.int32(0), unroll=8)

        def s1(i, mxi):
            v = row[pl.ds(i * 16, 16)]
            k = _sortable(v)
            bk = _bc(lax.shift_right_logical(k, _vfull(24, jnp.uint32)),
                     jnp.int32)
            plsc.addupdate_scatter(hist, [bk * 16 + lane], fones)
            return jnp.maximum(
                mxi, _bc(k ^ _vfull(_HI, jnp.uint32), jnp.int32))

        mxi = lax.fori_loop(
            0, _NV, s1, jnp.full((16,), jnp.iinfo(jnp.int32).min, jnp.int32),
            unroll=5)
        kmx = _bc(jnp.broadcast_to(jnp.max(mxi), (16,)),
                  jnp.uint32) ^ _vfull(_HI, jnp.uint32)
        m_v = _unsortable(kmx)

        trv = plsc.load_gather(tvm, [jnp.broadcast_to(r, (16,))])
        posv = plsc.load_gather(row, [trv])
        keyt = _sortable(posv)

        bmax = jnp.max(_bc(lax.shift_right_logical(kmx, _vfull(24, jnp.uint32)),
                           jnp.int32))

        def scan_down(need):
            def cond(car):
                b, cum, cab = car
                return jnp.logical_and(cum < need, b >= 0)

            def body(car):
                b, cum, cab = car
                s = jnp.sum(hist[pl.ds(b * 16, 16)])
                return (b - 1, cum + s, cum)

            b, _, cab = lax.while_loop(
                cond, body, (bmax, jnp.float32(0), jnp.float32(0)))
            return b + 1, cab

        b1, catot = scan_down(jnp.float32(_RANK))
        b1v = jnp.broadcast_to(b1, (16,)).astype(jnp.uint32)
        need = jnp.float32(_RANK) - catot

        def s3(i, car):
            w, e = car
            v = row[pl.ds(i * 16, 16)]
            k = _sortable(v)
            top8 = lax.shift_right_logical(k, _vfull(24, jnp.uint32))
            above = top8 > b1v
            eq = top8 == b1v
            e = e + jnp.where(above, jnp.exp(_SCALE * (v - m_v)), 0.0)
            mi = eq.astype(jnp.int32)
            incl = plsc.cumsum(mi)
            plsc.store_scatter(row, [w + incl - mi], v, mask=eq)
            return (w + jnp.sum(mi), e)

        w, e1 = lax.fori_loop(
            0, _NV, s3, (jnp.int32(0), jnp.zeros((16,), jnp.float32)),
            unroll=5)
        nv2 = lax.div(w + 15, jnp.int32(16))

        prefv = b1v
        for shift in (16, 8, 0):
            lax.fori_loop(0, 256, zb, jnp.int32(0), unroll=8)

            def s4(i, _, shift=shift, prefv=prefv):
                v = row[pl.ds(i * 16, 16)]
                k = _sortable(v)
                valid = (i * 16 + lane) < w
                match = lax.shift_right_logical(
                    k, _vfull(shift + 8, jnp.uint32)) == prefv
                bk = _bc(lax.shift_right_logical(k, _vfull(shift, jnp.uint32))
                         & _vfull(0xFF, jnp.uint32), jnp.int32)
                plsc.addupdate_scatter(
                    hist, [bk * 16 + lane], fones,
                    mask=jnp.logical_and(valid, match))
                return jnp.int32(0)

            lax.fori_loop(0, nv2, s4, jnp.int32(0))

            def cond2(car, need=need):
                b, cum, cab = car
                return jnp.logical_and(cum < need, b >= 0)

            def body2(car):
                b, cum, cab = car
                s = jnp.sum(hist[pl.ds(b * 16, 16)])
                return (b - 1, cum + s, cum)

            bafter, _, cab = lax.while_loop(
                cond2, body2, (jnp.int32(255), jnp.float32(0), jnp.float32(0)))
            bsel = bafter + 1
            prefv = (lax.shift_left(prefv, _vfull(8, jnp.uint32))
                     | jnp.broadcast_to(bsel, (16,)).astype(jnp.uint32))
            need = need - cab
            catot = catot + cab

        vprime = prefv

        def s5(i, e):
            v = row[pl.ds(i * 16, 16)]
            k = _sortable(v)
            gt = jnp.logical_and(k > vprime, (i * 16 + lane) < w)
            return e + jnp.where(gt, jnp.exp(_SCALE * (v - m_v)), 0.0)

        e2 = lax.fori_loop(0, nv2, s5, jnp.zeros((16,), jnp.float32))

        eab = jnp.broadcast_to(jnp.sum(e1) + jnp.sum(e2), (16,))
        tgtf = jnp.where(keyt > vprime, 1.0, 0.0)
        epos = jnp.exp(_SCALE * (posv - m_v))
        vpf = _unsortable(vprime)
        catf = jnp.broadcast_to(catot, (16,))
        sumexp = (eab - tgtf * epos
                  + (_vfull(float(_NEG), jnp.float32) - catf + tgtf)
                  * jnp.exp(_SCALE * (vpf - m_v))
                  + epos)
        outv = jnp.where(lane == 3 * j, sumexp, outv)
        outv = jnp.where(lane == 3 * j + 1, m_v, outv)
        outv = jnp.where(lane == 3 * j + 2, posv, outv)

    outb[...] = outv
    pltpu.sync_copy(outb, out_hbm.at[wid])


_sc_call = functools.partial(
    pl.kernel,
    out_type=jax.ShapeDtypeStruct((_NW, 16), jnp.float32),
    mesh=plsc.VectorSubcoreMesh(core_axis_name="c", subcore_axis_name="s"),
    compiler_params=pltpu.CompilerParams(needs_layout_passes=False),
    scratch_types=[
        pltpu.VMEM((_C,), jnp.float32),
        pltpu.VMEM((256 * 16,), jnp.float32),
        pltpu.VMEM((_B,), jnp.int32),
        pltpu.VMEM((16,), jnp.float32),
    ],
)(_sc_body)


def _fin_body(a_ref, o_ref):
    a = a_ref[...]
    tot = jnp.float32(0.0)
    for j in range(_RPW):
        se = a[:, 3 * j:3 * j + 1]
        m = a[:, 3 * j + 1:3 * j + 2]
        ps = a[:, 3 * j + 2:3 * j + 3]
        tot = tot + jnp.sum(_SCALE * m + jnp.log(se) - _SCALE * ps)
    o_ref[0, 0] = tot * (1.0 / _B)


@jax.jit
def kernel(logits, targets):
    parts = _sc_call(logits, targets.astype(jnp.int32))
    out = pl.pallas_call(
        _fin_body,
        out_specs=pl.BlockSpec(memory_space=pltpu.SMEM),
        out_shape=jax.ShapeDtypeStruct((1, 1), jnp.float32),
    )(parts)
    return out[0, 0]


# speedup vs baseline: 1.0472x; 1.0472x over previous
import functools

import jax
import jax.numpy as jnp
import numpy as np
from jax import lax
from jax.experimental import pallas as pl
from jax.experimental.pallas import tpu as pltpu
from jax.experimental.pallas import tpu_sc as plsc

_SCALE = 10.0
_B = 64
_C = 100000
_NEG = int(0.01 * (_C - 1))
_RANK = _NEG + 1
_NC = 2
_NS = 16
_L = 16
_NW = _NC * _NS
_RPW = _B // _NW
_NV = _C // _L
_HI = np.uint32(0x80000000)


def _bc(x, dt):
    return lax.bitcast_convert_type(x, dt)


def _vfull(val, dt):
    return jnp.full((16,), val, dt)


def _sortable(v):
    bu = _bc(v, jnp.uint32)
    asr = lax.shift_right_arithmetic(_bc(v, jnp.int32), _vfull(31, jnp.int32))
    flip = _bc(asr, jnp.uint32) | _vfull(_HI, jnp.uint32)
    return bu ^ flip


def _unsortable(k):
    b = jnp.where(k >= _vfull(_HI, jnp.uint32), k ^ _vfull(_HI, jnp.uint32), ~k)
    return _bc(b, jnp.float32)


def _sc_body(x_hbm, t_hbm, out_hbm, row, hist, tvm, outb):
    cid = lax.axis_index("c")
    sid = lax.axis_index("s")
    wid = sid * _NC + cid
    lane = lax.iota(jnp.int32, 16)
    fones = jnp.ones((16,), jnp.float32)
    fzeros = jnp.zeros((16,), jnp.float32)

    pltpu.sync_copy(t_hbm, tvm)

    def zb(i, _):
        hist[pl.ds(i * 16, 16)] = fzeros
        return jnp.int32(0)

    outv = jnp.zeros((16,), jnp.float32)

    for j in range(_RPW):
        r = wid * _RPW + j
        pltpu.sync_copy(x_hbm.at[r], row)

        lax.fori_loop(0, 256, zb, jnp.int32(0), unroll=8)

        def s1(i, mxi):
            v = row[pl.ds(i * 16, 16)]
            k = _sortable(v)
            bk = _bc(lax.shift_right_logical(k, _vfull(24, jnp.uint32)),
                     jnp.int32)
            plsc.addupdate_scatter(hist, [bk * 16 + lane], fones)
            return jnp.maximum(
                mxi, _bc(k ^ _vfull(_HI, jnp.uint32), jnp.int32))

        mxi = lax.fori_loop(
            0, _NV, s1, jnp.full((16,), jnp.iinfo(jnp.int32).min, jnp.int32),
            unroll=5)
        kmx = _bc(jnp.broadcast_to(jnp.max(mxi), (16,)),
                  jnp.uint32) ^ _vfull(_HI, jnp.uint32)
        m_v = _unsortable(kmx)

        trv = plsc.load_gather(tvm, [jnp.broadcast_to(r, (16,))])
        posv = plsc.load_gather(row, [trv])
        keyt = _sortable(posv)

        bmax = jnp.max(_bc(lax.shift_right_logical(kmx, _vfull(24, jnp.uint32)),
                           jnp.int32))

        def scan_down(need):
            def cond(car):
                b, cum, cab = car
                return jnp.logical_and(cum < need, b >= 0)

            def body(car):
                b, cum, cab = car
                s = jnp.sum(hist[pl.ds(b * 16, 16)])
                return (b - 1, cum + s, cum)

            b, _, cab = lax.while_loop(
                cond, body, (bmax, jnp.float32(0), jnp.float32(0)))
            return b + 1, cab

        b1, catot = scan_down(jnp.float32(_RANK))
        b1v = jnp.broadcast_to(b1, (16,)).astype(jnp.uint32)
        need = jnp.float32(_RANK) - catot

        def s3(i, car):
            wv, e = car
            v = row[pl.ds(i * 16, 16)]
            k = _sortable(v)
            top8 = lax.shift_right_logical(k, _vfull(24, jnp.uint32))
            above = top8 > b1v
            eq = top8 == b1v
            e = e + jnp.where(above, jnp.exp(_SCALE * (v - m_v)), 0.0)
            mi = eq.astype(jnp.int32)
            incl = plsc.cumsum(mi)
            plsc.store_scatter(row, [wv + incl - mi], v, mask=eq)
            return (wv + plsc.all_reduce_population_count(eq), e)

        wvec, e1 = lax.fori_loop(
            0, _NV, s3, (jnp.zeros((16,), jnp.int32),
                         jnp.zeros((16,), jnp.float32)),
            unroll=5)
        w = jnp.max(wvec)
        nv2 = lax.div(w + 15, jnp.int32(16))

        prefv = b1v
        for shift in (16, 8, 0):
            lax.fori_loop(0, 256, zb, jnp.int32(0), unroll=8)

            def s4(i, _, shift=shift, prefv=prefv):
                v = row[pl.ds(i * 16, 16)]
                k = _sortable(v)
                valid = (i * 16 + lane) < w
                match = lax.shift_right_logical(
                    k, _vfull(shift + 8, jnp.uint32)) == prefv
                bk = _bc(lax.shift_right_logical(k, _vfull(shift, jnp.uint32))
                         & _vfull(0xFF, jnp.uint32), jnp.int32)
                plsc.addupdate_scatter(
                    hist, [bk * 16 + lane], fones,
                    mask=jnp.logical_and(valid, match))
                return jnp.int32(0)

            lax.fori_loop(0, nv2, s4, jnp.int32(0))

            def cond2(car, need=need):
                b, cum, cab = car
                return jnp.logical_and(cum < need, b >= 0)

            def body2(car):
                b, cum, cab = car
                s = jnp.sum(hist[pl.ds(b * 16, 16)])
                return (b - 1, cum + s, cum)

            bafter, _, cab = lax.while_loop(
                cond2, body2, (jnp.int32(255), jnp.float32(0), jnp.float32(0)))
            bsel = bafter + 1
            prefv = (lax.shift_left(prefv, _vfull(8, jnp.uint32))
                     | jnp.broadcast_to(bsel, (16,)).astype(jnp.uint32))
            need = need - cab
            catot = catot + cab

        vprime = prefv

        def s5(i, e):
            v = row[pl.ds(i * 16, 16)]
            k = _sortable(v)
            gt = jnp.logical_and(k > vprime, (i * 16 + lane) < w)
            return e + jnp.where(gt, jnp.exp(_SCALE * (v - m_v)), 0.0)

        e2 = lax.fori_loop(0, nv2, s5, jnp.zeros((16,), jnp.float32))

        eab = jnp.broadcast_to(jnp.sum(e1) + jnp.sum(e2), (16,))
        tgtf = jnp.where(keyt > vprime, 1.0, 0.0)
        epos = jnp.exp(_SCALE * (posv - m_v))
        vpf = _unsortable(vprime)
        catf = jnp.broadcast_to(catot, (16,))
        sumexp = (eab - tgtf * epos
                  + (_vfull(float(_NEG), jnp.float32) - catf + tgtf)
                  * jnp.exp(_SCALE * (vpf - m_v))
                  + epos)
        outv = jnp.where(lane == 3 * j, sumexp, outv)
        outv = jnp.where(lane == 3 * j + 1, m_v, outv)
        outv = jnp.where(lane == 3 * j + 2, posv, outv)

    outb[...] = outv
    pltpu.sync_copy(outb, out_hbm.at[wid])


_sc_call = functools.partial(
    pl.kernel,
    out_type=jax.ShapeDtypeStruct((_NW, 16), jnp.float32),
    mesh=plsc.VectorSubcoreMesh(core_axis_name="c", subcore_axis_name="s"),
    compiler_params=pltpu.CompilerParams(needs_layout_passes=False),
    scratch_types=[
        pltpu.VMEM((_C,), jnp.float32),
        pltpu.VMEM((256 * 16,), jnp.float32),
        pltpu.VMEM((_B,), jnp.int32),
        pltpu.VMEM((16,), jnp.float32),
    ],
)(_sc_body)


def _fin_body(a_ref, o_ref):
    a = a_ref[...]
    tot = jnp.float32(0.0)
    for j in range(_RPW):
        se = a[:, 3 * j:3 * j + 1]
        m = a[:, 3 * j + 1:3 * j + 2]
        ps = a[:, 3 * j + 2:3 * j + 3]
        tot = tot + jnp.sum(_SCALE * m + jnp.log(se) - _SCALE * ps)
    o_ref[0, 0] = tot * (1.0 / _B)


@jax.jit
def kernel(logits, targets):
    parts = _sc_call(logits, targets.astype(jnp.int32))
    out = pl.pallas_call(
        _fin_body,
        out_specs=pl.BlockSpec(memory_space=pltpu.SMEM),
        out_shape=jax.ShapeDtypeStruct((1, 1), jnp.float32),
    )(parts)
    return out[0, 0]


# speedup vs baseline: 2.9634x; 2.8298x over previous
import functools

import jax
import jax.numpy as jnp
import numpy as np
from jax import lax
from jax.experimental import pallas as pl
from jax.experimental.pallas import tpu as pltpu
from jax.experimental.pallas import tpu_sc as plsc

_SCALE = 10.0
_B = 64
_C = 100000
_NEG = int(0.01 * (_C - 1))
_RANK = _NEG + 1
_NC = 2
_NS = 16
_L = 16
_NW = _NC * _NS
_RPW = _B // _NW
_NV = _C // _L
_HI = np.uint32(0x80000000)


def _bc(x, dt):
    return lax.bitcast_convert_type(x, dt)


def _vfull(val, dt):
    return jnp.full((16,), val, dt)


def _sortable(v):
    bu = _bc(v, jnp.uint32)
    asr = lax.shift_right_arithmetic(_bc(v, jnp.int32), _vfull(31, jnp.int32))
    flip = _bc(asr, jnp.uint32) | _vfull(_HI, jnp.uint32)
    return bu ^ flip


def _unsortable(k):
    b = jnp.where(k >= _vfull(_HI, jnp.uint32), k ^ _vfull(_HI, jnp.uint32), ~k)
    return _bc(b, jnp.float32)


def _sc_body(x_hbm, t_hbm, out_hbm, row, hist, tvm, outb):
    cid = lax.axis_index("c")
    sid = lax.axis_index("s")
    wid = sid * _NC + cid
    lane = lax.iota(jnp.int32, 16)
    fones = jnp.ones((16,), jnp.float32)
    fzeros = jnp.zeros((16,), jnp.float32)

    pltpu.sync_copy(t_hbm, tvm)

    def zb(i, _):
        hist[pl.ds(i * 16, 16)] = fzeros
        return jnp.int32(0)

    outv = jnp.zeros((16,), jnp.float32)

    for j in range(_RPW):
        r = wid * _RPW + j
        pltpu.sync_copy(x_hbm.at[r], row)

        lax.fori_loop(0, 256, zb, jnp.int32(0), unroll=8)

        @plsc.parallel_loop(0, _NV, unroll=5)
        def s1(i):
            v = row[pl.ds(i * 16, 16)]
            k = _sortable(v)
            bk = _bc(lax.shift_right_logical(k, _vfull(24, jnp.uint32)),
                     jnp.int32)
            plsc.addupdate_scatter(hist, [bk * 16 + lane], fones)

        trv = plsc.load_gather(tvm, [jnp.broadcast_to(r, (16,))])
        posv = plsc.load_gather(row, [trv])
        keyt = _sortable(posv)

        bmax = jnp.int32(255)

        def scan_down(need):
            def cond(car):
                b, cum, cab = car
                return jnp.logical_and(cum < need, b >= 0)

            def body(car):
                b, cum, cab = car
                s = jnp.sum(hist[pl.ds(b * 16, 16)])
                return (b - 1, cum + s, cum)

            b, _, cab = lax.while_loop(
                cond, body, (bmax, jnp.float32(0), jnp.float32(0)))
            return b + 1, cab

        b1, catot = scan_down(jnp.float32(_RANK))
        b1v = jnp.broadcast_to(b1, (16,)).astype(jnp.uint32)
        need = jnp.float32(_RANK) - catot

        @plsc.parallel_loop(
            0, _NV, carry=jnp.zeros((16,), jnp.int32), unroll=5)
        def s3(i, wv):
            v = row[pl.ds(i * 16, 16)]
            k = _sortable(v)
            ge = lax.shift_right_logical(k, _vfull(24, jnp.uint32)) >= b1v
            mi = ge.astype(jnp.int32)
            incl = plsc.cumsum(mi)
            plsc.store_scatter(row, [wv + incl - mi], v, mask=ge)
            return wv + plsc.all_reduce_population_count(ge)

        w = jnp.max(s3)
        nv2 = lax.div(w + 15, jnp.int32(16))

        def smx(i, mxi):
            v = row[pl.ds(i * 16, 16)]
            ki = _bc(_sortable(v) ^ _vfull(_HI, jnp.uint32), jnp.int32)
            valid = (i * 16 + lane) < w
            return jnp.maximum(
                mxi, jnp.where(valid, ki,
                               jnp.full((16,), jnp.iinfo(jnp.int32).min,
                                        jnp.int32)))

        mxi = lax.fori_loop(
            0, nv2, smx, jnp.full((16,), jnp.iinfo(jnp.int32).min, jnp.int32))
        kmx = _bc(jnp.broadcast_to(jnp.max(mxi), (16,)),
                  jnp.uint32) ^ _vfull(_HI, jnp.uint32)
        m_v = _unsortable(kmx)

        prefv = b1v
        for shift in (16, 8, 0):
            lax.fori_loop(0, 256, zb, jnp.int32(0), unroll=8)

            def s4(i, _, shift=shift, prefv=prefv):
                v = row[pl.ds(i * 16, 16)]
                k = _sortable(v)
                valid = (i * 16 + lane) < w
                match = lax.shift_right_logical(
                    k, _vfull(shift + 8, jnp.uint32)) == prefv
                bk = _bc(lax.shift_right_logical(k, _vfull(shift, jnp.uint32))
                         & _vfull(0xFF, jnp.uint32), jnp.int32)
                plsc.addupdate_scatter(
                    hist, [bk * 16 + lane], fones,
                    mask=jnp.logical_and(valid, match))
                return jnp.int32(0)

            lax.fori_loop(0, nv2, s4, jnp.int32(0))

            def cond2(car, need=need):
                b, cum, cab = car
                return jnp.logical_and(cum < need, b >= 0)

            def body2(car):
                b, cum, cab = car
                s = jnp.sum(hist[pl.ds(b * 16, 16)])
                return (b - 1, cum + s, cum)

            bafter, _, cab = lax.while_loop(
                cond2, body2, (jnp.int32(255), jnp.float32(0), jnp.float32(0)))
            bsel = bafter + 1
            prefv = (lax.shift_left(prefv, _vfull(8, jnp.uint32))
                     | jnp.broadcast_to(bsel, (16,)).astype(jnp.uint32))
            need = need - cab
            catot = catot + cab

        vprime = prefv

        def s5(i, e):
            v = row[pl.ds(i * 16, 16)]
            k = _sortable(v)
            gt = jnp.logical_and(k > vprime, (i * 16 + lane) < w)
            return e + jnp.where(gt, jnp.exp(_SCALE * (v - m_v)), 0.0)

        e2 = lax.fori_loop(0, nv2, s5, jnp.zeros((16,), jnp.float32))

        eab = jnp.broadcast_to(jnp.sum(e2), (16,))
        tgtf = jnp.where(keyt > vprime, 1.0, 0.0)
        epos = jnp.exp(_SCALE * (posv - m_v))
        vpf = _unsortable(vprime)
        catf = jnp.broadcast_to(catot, (16,))
        sumexp = (eab - tgtf * epos
                  + (_vfull(float(_NEG), jnp.float32) - catf + tgtf)
                  * jnp.exp(_SCALE * (vpf - m_v))
                  + epos)
        outv = jnp.where(lane == 3 * j, sumexp, outv)
        outv = jnp.where(lane == 3 * j + 1, m_v, outv)
        outv = jnp.where(lane == 3 * j + 2, posv, outv)

    outb[...] = outv
    pltpu.sync_copy(outb, out_hbm.at[wid])


_sc_call = functools.partial(
    pl.kernel,
    out_type=jax.ShapeDtypeStruct((_NW, 16), jnp.float32),
    mesh=plsc.VectorSubcoreMesh(core_axis_name="c", subcore_axis_name="s"),
    compiler_params=pltpu.CompilerParams(needs_layout_passes=False),
    scratch_types=[
        pltpu.VMEM((_C,), jnp.float32),
        pltpu.VMEM((256 * 16,), jnp.float32),
        pltpu.VMEM((_B,), jnp.int32),
        pltpu.VMEM((16,), jnp.float32),
    ],
)(_sc_body)


def _fin_body(a_ref, o_ref):
    a = a_ref[...]
    tot = jnp.float32(0.0)
    for j in range(_RPW):
        se = a[:, 3 * j:3 * j + 1]
        m = a[:, 3 * j + 1:3 * j + 2]
        ps = a[:, 3 * j + 2:3 * j + 3]
        tot = tot + jnp.sum(_SCALE * m + jnp.log(se) - _SCALE * ps)
    o_ref[0, 0] = tot * (1.0 / _B)


@jax.jit
def kernel(logits, targets):
    parts = _sc_call(logits, targets.astype(jnp.int32))
    out = pl.pallas_call(
        _fin_body,
        out_specs=pl.BlockSpec(memory_space=pltpu.SMEM),
        out_shape=jax.ShapeDtypeStruct((1, 1), jnp.float32),
    )(parts)
    return out[0, 0]


# speedup vs baseline: 3.1145x; 1.0510x over previous
import functools

import jax
import jax.numpy as jnp
import numpy as np
from jax import lax
from jax.experimental import pallas as pl
from jax.experimental.pallas import tpu as pltpu
from jax.experimental.pallas import tpu_sc as plsc

_SCALE = 10.0
_B = 64
_C = 100000
_NEG = int(0.01 * (_C - 1))
_RANK = _NEG + 1
_NC = 2
_NS = 16
_L = 16
_NW = _NC * _NS
_RPW = _B // _NW
_NV = _C // _L
_HI = np.uint32(0x80000000)


def _bc(x, dt):
    return lax.bitcast_convert_type(x, dt)


def _vfull(val, dt):
    return jnp.full((16,), val, dt)


def _sortable(v):
    bu = _bc(v, jnp.uint32)
    asr = lax.shift_right_arithmetic(_bc(v, jnp.int32), _vfull(31, jnp.int32))
    flip = _bc(asr, jnp.uint32) | _vfull(_HI, jnp.uint32)
    return bu ^ flip


def _unsortable(k):
    b = jnp.where(k >= _vfull(_HI, jnp.uint32), k ^ _vfull(_HI, jnp.uint32), ~k)
    return _bc(b, jnp.float32)


def _sc_body(x_hbm, t_hbm, out_hbm, row, hist, tvm, outb):
    cid = lax.axis_index("c")
    sid = lax.axis_index("s")
    wid = sid * _NC + cid
    lane = lax.iota(jnp.int32, 16)
    fones = jnp.ones((16,), jnp.float32)
    fzeros = jnp.zeros((16,), jnp.float32)

    pltpu.sync_copy(t_hbm, tvm)

    def zb(i, _):
        hist[pl.ds(i * 16, 16)] = fzeros
        return jnp.int32(0)

    outv = jnp.zeros((16,), jnp.float32)

    for j in range(_RPW):
        r = wid * _RPW + j
        pltpu.sync_copy(x_hbm.at[r], row)

        lax.fori_loop(0, 256, zb, jnp.int32(0), unroll=8)

        @plsc.parallel_loop(0, _NV, unroll=10)
        def s1(i):
            v = row[pl.ds(i * 16, 16)]
            k = _sortable(v)
            bk = _bc(lax.shift_right_logical(k, _vfull(24, jnp.uint32)),
                     jnp.int32)
            plsc.addupdate_scatter(hist, [bk * 16 + lane], fones)

        trv = plsc.load_gather(tvm, [jnp.broadcast_to(r, (16,))])
        posv = plsc.load_gather(row, [trv])
        keyt = _sortable(posv)

        bmax = jnp.int32(255)

        def scan_down(need):
            def cond(car):
                b, cum, cab = car
                return jnp.logical_and(cum < need, b >= 0)

            def body(car):
                b, cum, cab = car
                s = jnp.sum(hist[pl.ds(b * 16, 16)])
                return (b - 1, cum + s, cum)

            b, _, cab = lax.while_loop(
                cond, body, (bmax, jnp.float32(0), jnp.float32(0)))
            return b + 1, cab

        b1, catot = scan_down(jnp.float32(_RANK))
        b1v = jnp.broadcast_to(b1, (16,)).astype(jnp.uint32)
        need = jnp.float32(_RANK) - catot

        @plsc.parallel_loop(
            0, _NV, carry=jnp.zeros((16,), jnp.int32), unroll=10)
        def s3(i, wv):
            v = row[pl.ds(i * 16, 16)]
            k = _sortable(v)
            ge = lax.shift_right_logical(k, _vfull(24, jnp.uint32)) >= b1v
            mi = ge.astype(jnp.int32)
            incl = plsc.cumsum(mi)
            plsc.store_scatter(row, [wv + incl - mi], v, mask=ge)
            return wv + plsc.all_reduce_population_count(ge)

        w = jnp.max(s3)
        nv2 = lax.div(w + 15, jnp.int32(16))

        def smx(i, mxi):
            v = row[pl.ds(i * 16, 16)]
            ki = _bc(_sortable(v) ^ _vfull(_HI, jnp.uint32), jnp.int32)
            valid = (i * 16 + lane) < w
            return jnp.maximum(
                mxi, jnp.where(valid, ki,
                               jnp.full((16,), jnp.iinfo(jnp.int32).min,
                                        jnp.int32)))

        mxi = lax.fori_loop(
            0, nv2, smx, jnp.full((16,), jnp.iinfo(jnp.int32).min, jnp.int32))
        kmx = _bc(jnp.broadcast_to(jnp.max(mxi), (16,)),
                  jnp.uint32) ^ _vfull(_HI, jnp.uint32)
        m_v = _unsortable(kmx)

        prefv = b1v
        for shift in (16, 8, 0):
            lax.fori_loop(0, 256, zb, jnp.int32(0), unroll=8)

            def s4(i, _, shift=shift, prefv=prefv):
                v = row[pl.ds(i * 16, 16)]
                k = _sortable(v)
                valid = (i * 16 + lane) < w
                match = lax.shift_right_logical(
                    k, _vfull(shift + 8, jnp.uint32)) == prefv
                bk = _bc(lax.shift_right_logical(k, _vfull(shift, jnp.uint32))
                         & _vfull(0xFF, jnp.uint32), jnp.int32)
                plsc.addupdate_scatter(
                    hist, [bk * 16 + lane], fones,
                    mask=jnp.logical_and(valid, match))
                return jnp.int32(0)

            lax.fori_loop(0, nv2, s4, jnp.int32(0))

            def cond2(car, need=need):
                b, cum, cab = car
                return jnp.logical_and(cum < need, b >= 0)

            def body2(car):
                b, cum, cab = car
                s = jnp.sum(hist[pl.ds(b * 16, 16)])
                return (b - 1, cum + s, cum)

            bafter, _, cab = lax.while_loop(
                cond2, body2, (jnp.int32(255), jnp.float32(0), jnp.float32(0)))
            bsel = bafter + 1
            prefv = (lax.shift_left(prefv, _vfull(8, jnp.uint32))
                     | jnp.broadcast_to(bsel, (16,)).astype(jnp.uint32))
            need = need - cab
            catot = catot + cab

        vprime = prefv

        def s5(i, e):
            v = row[pl.ds(i * 16, 16)]
            k = _sortable(v)
            gt = jnp.logical_and(k > vprime, (i * 16 + lane) < w)
            return e + jnp.where(gt, jnp.exp(_SCALE * (v - m_v)), 0.0)

        e2 = lax.fori_loop(0, nv2, s5, jnp.zeros((16,), jnp.float32))

        eab = jnp.broadcast_to(jnp.sum(e2), (16,))
        tgtf = jnp.where(keyt > vprime, 1.0, 0.0)
        epos = jnp.exp(_SCALE * (posv - m_v))
        vpf = _unsortable(vprime)
        catf = jnp.broadcast_to(catot, (16,))
        sumexp = (eab - tgtf * epos
                  + (_vfull(float(_NEG), jnp.float32) - catf + tgtf)
                  * jnp.exp(_SCALE * (vpf - m_v))
                  + epos)
        outv = jnp.where(lane == 3 * j, sumexp, outv)
        outv = jnp.where(lane == 3 * j + 1, m_v, outv)
        outv = jnp.where(lane == 3 * j + 2, posv, outv)

    outb[...] = outv
    pltpu.sync_copy(outb, out_hbm.at[wid])


_sc_call = functools.partial(
    pl.kernel,
    out_type=jax.ShapeDtypeStruct((_NW, 16), jnp.float32),
    mesh=plsc.VectorSubcoreMesh(core_axis_name="c", subcore_axis_name="s"),
    compiler_params=pltpu.CompilerParams(needs_layout_passes=False),
    scratch_types=[
        pltpu.VMEM((_C,), jnp.float32),
        pltpu.VMEM((256 * 16,), jnp.float32),
        pltpu.VMEM((_B,), jnp.int32),
        pltpu.VMEM((16,), jnp.float32),
    ],
)(_sc_body)


def _fin_body(a_ref, o_ref):
    a = a_ref[...]
    tot = jnp.float32(0.0)
    for j in range(_RPW):
        se = a[:, 3 * j:3 * j + 1]
        m = a[:, 3 * j + 1:3 * j + 2]
        ps = a[:, 3 * j + 2:3 * j + 3]
        tot = tot + jnp.sum(_SCALE * m + jnp.log(se) - _SCALE * ps)
    o_ref[0, 0] = tot * (1.0 / _B)


@jax.jit
def kernel(logits, targets):
    parts = _sc_call(logits, targets.astype(jnp.int32))
    out = pl.pallas_call(
        _fin_body,
        out_specs=pl.BlockSpec(memory_space=pltpu.SMEM),
        out_shape=jax.ShapeDtypeStruct((1, 1), jnp.float32),
    )(parts)
    return out[0, 0]
